# baseline (speedup 1.0000x reference)
import functools

import jax
import jax.numpy as jnp
from jax import lax
from jax.experimental import pallas as pl
from jax.experimental.pallas import tpu as pltpu
from jax.experimental.pallas import tpu_sc as plsc

VOCAB = 1000
VPAD = 1024
BATCH = 16384

_info = plsc.get_sparse_core_info()
NC, NS = _info.num_cores, _info.num_subcores
NW = NC * NS
B_PER_W = BATCH // NW
R = 16
CH = B_PER_W // R


def _gather_kernel(table_pad, idx2):
    mesh = plsc.VectorSubcoreMesh(core_axis_name="c", subcore_axis_name="s")

    @functools.partial(
        pl.kernel,
        mesh=mesh,
        out_type=jax.ShapeDtypeStruct((BATCH, VPAD), jnp.float32),
        scratch_types=[
            pltpu.VMEM((CH, R), jnp.int32),
            pltpu.VMEM((R, VPAD), jnp.float32),
            pltpu.VMEM((R, VPAD), jnp.float32),
            pltpu.SemaphoreType.DMA,
            pltpu.SemaphoreType.DMA,
            pltpu.SemaphoreType.DMA,
            pltpu.SemaphoreType.DMA,
        ],
    )
    def k(table_hbm, idx_hbm, out_hbm, idx_v, buf_a, buf_b, ga, gb, wa, wb):
        sid = lax.axis_index("s")
        wid = sid * NC + lax.axis_index("c")
        base = wid * B_PER_W
        pltpu.sync_copy(idx_hbm.at[pl.ds(wid * CH, CH)], idx_v)

        bufs = (buf_a, buf_b)
        gsems = (ga, gb)
        wsems = (wa, wb)

        def gather(c, b):
            return pltpu.async_copy(
                table_hbm.at[idx_v.at[c]], bufs[b], gsems[b]
            )

        def write(c, b):
            return pltpu.async_copy(
                bufs[b], out_hbm.at[pl.ds(base + c * R, R)], wsems[b]
            )

        gh = [gather(0, 0), None]
        wh = [None, None]
        for c in range(CH):
            b = c % 2
            nb = 1 - b
            if c + 1 < CH:
                if wh[nb] is not None:
                    wh[nb].wait()
                gh[nb] = gather(c + 1, nb)
            gh[b].wait()
            wh[b] = write(c, b)
        wh[0].wait()
        wh[1].wait()

    return k(table_pad, idx2)


_UNPAD_ROWS = 512


def _unpad_body(inp_ref, out_ref):
    out_ref[...] = inp_ref[:, :VOCAB]


def _unpad(out_pad):
    return pl.pallas_call(
        _unpad_body,
        grid=(BATCH // _UNPAD_ROWS,),
        in_specs=[pl.BlockSpec((_UNPAD_ROWS, VPAD), lambda i: (i, 0))],
        out_specs=pl.BlockSpec((_UNPAD_ROWS, VOCAB), lambda i: (i, 0)),
        out_shape=jax.ShapeDtypeStruct((BATCH, VOCAB), jnp.float32),
    )(out_pad)


def kernel(idx, token_embedding_table):
    table_pad = jnp.pad(token_embedding_table, ((0, 0), (0, VPAD - VOCAB)))
    idx2 = idx.reshape(NW * CH, R)
    out_pad = _gather_kernel(table_pad, idx2)
    return _unpad(out_pad)


# speedup vs baseline: 1.4690x; 1.4690x over previous
import functools

import jax
import jax.numpy as jnp
from jax import lax
from jax.experimental import pallas as pl
from jax.experimental.pallas import tpu as pltpu
from jax.experimental.pallas import tpu_sc as plsc

VOCAB = 1000
VPAD = 1024
BATCH = 16384

_info = plsc.get_sparse_core_info()
NC, NS = _info.num_cores, _info.num_subcores
NW = NC * NS
B_PER_W = BATCH // NW
R = 32
CH = B_PER_W // R


def _gather_kernel(table_pad, idx2):
    mesh = plsc.VectorSubcoreMesh(core_axis_name="c", subcore_axis_name="s")

    @functools.partial(
        pl.kernel,
        mesh=mesh,
        out_type=jax.ShapeDtypeStruct((BATCH, VPAD), jnp.float32),
        scratch_types=[
            pltpu.VMEM((CH, R), jnp.int32),
            pltpu.VMEM((R, VPAD), jnp.float32),
            pltpu.VMEM((R, VPAD), jnp.float32),
            pltpu.SemaphoreType.DMA,
            pltpu.SemaphoreType.DMA,
            pltpu.SemaphoreType.DMA,
            pltpu.SemaphoreType.DMA,
        ],
    )
    def k(table_hbm, idx_hbm, out_hbm, idx_v, buf_a, buf_b, ga, gb, wa, wb):
        sid = lax.axis_index("s")
        wid = sid * NC + lax.axis_index("c")
        base = wid * B_PER_W
        pltpu.sync_copy(idx_hbm.at[pl.ds(wid * CH, CH)], idx_v)

        bufs = (buf_a, buf_b)
        gsems = (ga, gb)
        wsems = (wa, wb)

        def gather(c, b):
            return pltpu.async_copy(
                table_hbm.at[idx_v.at[c]], bufs[b], gsems[b]
            )

        def write(c, b):
            return pltpu.async_copy(
                bufs[b], out_hbm.at[pl.ds(base + c * R, R)], wsems[b]
            )

        gh = [gather(0, 0), None]
        wh = [None, None]
        for c in range(CH):
            b = c % 2
            nb = 1 - b
            if c + 1 < CH:
                if wh[nb] is not None:
                    wh[nb].wait()
                gh[nb] = gather(c + 1, nb)
            gh[b].wait()
            wh[b] = write(c, b)
        wh[0].wait()
        wh[1].wait()

    return k(table_pad, idx2)


def kernel(idx, token_embedding_table):
    table_pad = jnp.pad(token_embedding_table, ((0, 0), (0, VPAD - VOCAB)))
    idx2 = idx.reshape(NW * CH, R)
    out_pad = _gather_kernel(table_pad, idx2)
    return out_pad[:, :VOCAB]
